# Initial kernel scaffold; baseline (speedup 1.0000x reference)
import jax
import jax.numpy as jnp
from jax.experimental import pallas as pl


def kernel(points, sa1, sa2, sa3, glob, Wc, bc):
    raise NotImplementedError("write your pallas kernel here")


# speedup vs baseline: 14.1297x; 14.1297x over previous
import functools

import jax
import jax.numpy as jnp
from jax import lax
from jax.experimental import pallas as pl
from jax.experimental.pallas import tpu as pltpu
from jax.experimental.pallas import tpu_sc as plsc

B = 16
N = 4096
EPS = 1e-5
_SC_CHUNK = 128


def _fps_body(npoint, n, px, py, pz, ox, oy, oz):
    b = px.shape[0]
    lane = lax.broadcasted_iota(jnp.int32, (b, n), 1)
    slot = lax.broadcasted_iota(jnp.int32, (b, npoint), 1)
    x = px[...]
    y = py[...]
    z = pz[...]

    def body(i, state):
        oxv, oyv, ozv, distance, far = state
        sel = lane == far
        cx = jnp.sum(jnp.where(sel, x, 0.0), axis=1, keepdims=True)
        cy = jnp.sum(jnp.where(sel, y, 0.0), axis=1, keepdims=True)
        cz = jnp.sum(jnp.where(sel, z, 0.0), axis=1, keepdims=True)
        hit = slot == i
        oxv = jnp.where(hit, cx, oxv)
        oyv = jnp.where(hit, cy, oyv)
        ozv = jnp.where(hit, cz, ozv)
        d = (x - cx) ** 2 + (y - cy) ** 2 + (z - cz) ** 2
        distance = jnp.minimum(distance, d)
        m = jnp.max(distance, axis=1, keepdims=True)
        far = jnp.min(jnp.where(distance == m, lane, n), axis=1, keepdims=True)
        return oxv, oyv, ozv, distance, far

    init = (
        jnp.zeros((b, npoint), jnp.float32),
        jnp.zeros((b, npoint), jnp.float32),
        jnp.zeros((b, npoint), jnp.float32),
        jnp.full((b, n), 1e10, jnp.float32),
        jnp.zeros((b, 1), jnp.int32),
    )
    oxv, oyv, ozv, _, _ = lax.fori_loop(0, npoint, body, init)
    ox[...] = oxv
    oy[...] = oyv
    oz[...] = ozv


def _fps(px, py, pz, npoint):
    b, n = px.shape
    out = jax.ShapeDtypeStruct((b, npoint), jnp.float32)
    return pl.pallas_call(
        functools.partial(_fps_body, npoint, n),
        out_shape=[out, out, out],
    )(px, py, pz)


def _ballq_body(n, s, ns, r2, px, py, pz, cx, cy, cz, out):
    x = px[0]
    y = py[0]
    z = pz[0]
    ax = cx[0]
    ay = cy[0]
    az = cz[0]
    lane = lax.broadcasted_iota(jnp.int32, (s, n), 1)
    slot = lax.broadcasted_iota(jnp.int32, (s, ns), 1)
    a2 = (ax * ax + ay * ay) + az * az
    b2 = (x * x + y * y) + z * z
    zc = jnp.zeros((s, 5), jnp.bfloat16)
    amat = jnp.concatenate(
        [ax.astype(jnp.bfloat16), ay.astype(jnp.bfloat16),
         az.astype(jnp.bfloat16), zc], axis=1)
    bmat = jnp.concatenate(
        [x.astype(jnp.bfloat16), y.astype(jnp.bfloat16),
         z.astype(jnp.bfloat16), jnp.zeros((5, n), jnp.bfloat16)], axis=0)
    e = jax.lax.dot_general(
        amat, bmat, (((1,), (0,)), ((), ())),
        preferred_element_type=jnp.float32)
    d2 = (a2 - 2.0 * e) + b2
    masked = jnp.where(d2 <= r2, lane, n)

    def body(k, state):
        gidx, prev = state
        cand = jnp.where(masked > prev, masked, n)
        mn = jnp.min(cand, axis=1, keepdims=True)
        gidx = jnp.where(slot == k, mn, gidx)
        return gidx, mn

    gidx0 = jnp.full((s, ns), n, jnp.int32)
    prev0 = jnp.full((s, 1), -1, jnp.int32)
    gidx, _ = lax.fori_loop(0, ns, body, (gidx0, prev0))
    first = gidx[:, :1]
    gidx = jnp.where(gidx == n, first, gidx)
    gidx = jnp.minimum(gidx, n - 1)
    base = pl.program_id(0) * n
    out[...] = (gidx + base)[None]


def _ballq(px, py, pz, cx, cy, cz, radius, ns):
    b, n = px.shape
    s = cx.shape[1]
    cxT = cx.reshape(b, s, 1)
    cyT = cy.reshape(b, s, 1)
    czT = cz.reshape(b, s, 1)
    pspec = pl.BlockSpec((1, 1, n), lambda i: (i, 0, 0))
    cspec = pl.BlockSpec((1, s, 1), lambda i: (i, 0, 0))
    return pl.pallas_call(
        functools.partial(_ballq_body, n, s, ns, radius * radius),
        grid=(b,),
        in_specs=[pspec, pspec, pspec, cspec, cspec, cspec],
        out_specs=pl.BlockSpec((1, s, ns), lambda i: (i, 0, 0)),
        out_shape=jax.ShapeDtypeStruct((b, s, ns), jnp.int32),
    )(px.reshape(b, 1, n), py.reshape(b, 1, n), pz.reshape(b, 1, n),
      cxT, cyT, czT)


def _gather_rows(table, idx):
    t = idx.shape[0]
    d = table.shape[1]
    info = plsc.get_sparse_core_info()
    nw = info.num_cores * info.num_subcores
    per_w = t // nw
    n_chunks = per_w // _SC_CHUNK
    idx3 = idx.reshape(nw, n_chunks, _SC_CHUNK)
    mesh = plsc.VectorSubcoreMesh(core_axis_name="c", subcore_axis_name="s")

    @functools.partial(
        pl.kernel,
        mesh=mesh,
        compiler_params=pltpu.CompilerParams(use_tc_tiling_on_sc=True),
        out_type=jax.ShapeDtypeStruct((t, d), jnp.float32),
        scratch_types=[
            pltpu.VMEM((n_chunks, _SC_CHUNK), jnp.int32),
            pltpu.VMEM((_SC_CHUNK, d), jnp.float32),
            pltpu.SemaphoreType.DMA,
        ],
    )
    def k(table_hbm, idx_hbm, out_hbm, idx_v, rows_v, sem):
        wid = lax.axis_index("s") * info.num_cores + lax.axis_index("c")
        base = wid * per_w
        pltpu.sync_copy(idx_hbm.at[wid], idx_v)

        def body(c, carry):
            pltpu.async_copy(table_hbm.at[idx_v.at[c]], rows_v, sem).wait()
            pltpu.sync_copy(
                rows_v, out_hbm.at[pl.ds(base + c * _SC_CHUNK, _SC_CHUNK)]
            )
            return carry

        lax.fori_loop(0, n_chunks, body, 0)

    return k(table, idx3)


def _bf(x):
    return x.astype(jnp.bfloat16)


def _dotb(a, w):
    return jnp.dot(_bf(a), _bf(w), preferred_element_type=jnp.float32)


def _sa_tail_body(sn, ns, g, cx, cy, cz, w1, s1, b1, w2, s2, b2, w3, s3, b3,
                  out):
    cp = g.shape[2]
    gx = g[0].reshape(sn, ns, cp)
    ax = cx[0]
    ay = cy[0]
    az = cz[0]
    cvec = jnp.concatenate([ax, ay, az, jnp.zeros((sn, cp - 3), jnp.float32)],
                           axis=1)
    xin = (gx - cvec[:, None, :]).reshape(sn * ns, cp)
    h1 = jnp.maximum(s1[...] * _dotb(xin, w1[...]) + b1[...], 0.0)
    h2 = jnp.maximum(s2[...] * _dotb(h1, w2[...]) + b2[...], 0.0)
    h3 = jnp.maximum(s3[...] * _dotb(h2, w3[...]) + b3[...], 0.0)
    out[...] = jnp.max(h3.reshape(sn, ns, h3.shape[1]), axis=1)[None]


def _sa_tail(g, cx, cy, cz, w1, s1, b1, w2, s2, b2, w3, s3, b3):
    b, tot, cp = g.shape
    sn = cx.shape[1]
    ns = tot // sn
    c3 = w3.shape[1]
    cspec = pl.BlockSpec((1, sn, 1), lambda i: (i, 0, 0))
    full = lambda *shape: pl.BlockSpec(shape, lambda i: tuple(0 for _ in shape))
    return pl.pallas_call(
        functools.partial(_sa_tail_body, sn, ns),
        grid=(b,),
        in_specs=[
            pl.BlockSpec((1, tot, cp), lambda i: (i, 0, 0)),
            cspec, cspec, cspec,
            full(*w1.shape), full(*s1.shape), full(*b1.shape),
            full(*w2.shape), full(*s2.shape), full(*b2.shape),
            full(*w3.shape), full(*s3.shape), full(*b3.shape),
        ],
        out_specs=pl.BlockSpec((1, sn, c3), lambda i: (i, 0, 0)),
        out_shape=jax.ShapeDtypeStruct((b, sn, c3), jnp.float32),
    )(g, cx.reshape(b, sn, 1), cy.reshape(b, sn, 1), cz.reshape(b, sn, 1),
      w1, s1, b1, w2, s2, b2, w3, s3, b3)


def _head_body(f2, cx, cy, cz, w31, s31, b31, w32, s32, b32, w33, s33,
               b33, wg1, sg1, bg1, wg2, sg2, bg2, wc, bcv, out):
    f = f2[0]
    x = jnp.concatenate([cx[0], cy[0], cz[0], f], axis=1)
    x1 = jnp.maximum(s31[...] * _dotb(x, w31[...]) + b31[...], 0.0)
    x2 = jnp.maximum(s32[...] * _dotb(x1, w32[...]) + b32[...], 0.0)
    x3 = jnp.maximum(s33[...] * _dotb(x2, w33[...]) + b33[...], 0.0)
    v = jnp.max(x3, axis=0, keepdims=True)
    g1 = jnp.maximum(sg1[...] * _dotb(v, wg1[...]) + bg1[...], 0.0)
    g2 = jnp.maximum(sg2[...] * _dotb(g1, wg2[...]) + bg2[...], 0.0)
    out[0] = _dotb(g2, wc[...]) + bcv[...]


def _head(f2, cx, cy, cz, w31, s31, b31, w32, s32, b32, w33, s33, b33,
          wg1, sg1, bg1, wg2, sg2, bg2, wc, bcv):
    b, s2n, c0 = f2.shape
    nc = wc.shape[1]
    cspec = pl.BlockSpec((1, s2n, 1), lambda i: (i, 0, 0))
    full = lambda *shape: pl.BlockSpec(shape, lambda i: tuple(0 for _ in shape))
    return pl.pallas_call(
        _head_body,
        grid=(b,),
        in_specs=[
            pl.BlockSpec((1, s2n, c0), lambda i: (i, 0, 0)),
            cspec, cspec, cspec,
            full(*w31.shape), full(*s31.shape),
            full(*b31.shape), full(*w32.shape), full(*s32.shape),
            full(*b32.shape), full(*w33.shape), full(*s33.shape),
            full(*b33.shape), full(*wg1.shape), full(*sg1.shape),
            full(*bg1.shape), full(*wg2.shape), full(*sg2.shape),
            full(*bg2.shape), full(*wc.shape), full(*bcv.shape),
        ],
        out_specs=pl.BlockSpec((1, 1, nc), lambda i: (i, 0, 0)),
        out_shape=jax.ShapeDtypeStruct((b, 1, nc), jnp.float32),
    )(f2, cx.reshape(b, s2n, 1), cy.reshape(b, s2n, 1), cz.reshape(b, s2n, 1),
      w31, s31, b31, w32, s32, b32, w33, s33, b33,
      wg1, sg1, bg1, wg2, sg2, bg2, wc, bcv).reshape(b, nc)


def _scale(gamma):
    return (gamma / jnp.sqrt(jnp.float32(1.0 + EPS))).reshape(1, -1)


def kernel(points, sa1, sa2, sa3, glob, Wc, bc):
    (w11, g11, b11), (w12, g12, b12), (w13, g13, b13) = sa1
    (w21, g21, b21), (w22, g22, b22), (w23, g23, b23) = sa2
    (w31, g31, b31), (w32, g32, b32), (w33, g33, b33) = sa3
    (wg1, gg1, bg1), (wg2, gg2, bg2) = glob

    px = points[:, 0, :]
    py = points[:, 1, :]
    pz = points[:, 2, :]

    pad = lambda a, r, c: jnp.pad(a, ((0, r - a.shape[0]), (0, c - a.shape[1])))
    ox1, oy1, oz1 = _fps(px, py, pz, 512)
    gidx1 = _ballq(px, py, pz, ox1, oy1, oz1, 0.2, 32)
    table1 = jnp.pad(jnp.stack([px, py, pz], axis=-1), ((0, 0), (0, 0), (0, 125)))
    g1 = _gather_rows(table1.reshape(B * N, 128), gidx1.reshape(-1))
    feat1 = _sa_tail(
        g1.reshape(B, 512 * 32, 128), ox1, oy1, oz1,
        pad(w11, 128, 64), _scale(g11), b11.reshape(1, -1),
        w12, _scale(g12), b12.reshape(1, -1),
        w13, _scale(g13), b13.reshape(1, -1),
    )

    ox2, oy2, oz2 = _fps(ox1, oy1, oz1, 128)
    gidx2 = _ballq(ox1, oy1, oz1, ox2, oy2, oz2, 0.4, 64)
    table2 = jnp.concatenate(
        [jnp.stack([ox1, oy1, oz1], axis=-1), feat1,
         jnp.zeros((B, 512, 125), jnp.float32)], axis=-1)
    g2 = _gather_rows(table2.reshape(B * 512, 256), gidx2.reshape(-1))
    feat2 = _sa_tail(
        g2.reshape(B, 128 * 64, 256), ox2, oy2, oz2,
        pad(w21, 256, 128), _scale(g21), b21.reshape(1, -1),
        w22, _scale(g22), b22.reshape(1, -1),
        w23, _scale(g23), b23.reshape(1, -1),
    )

    logits = _head(
        feat2, ox2, oy2, oz2,
        w31, _scale(g31), b31.reshape(1, -1),
        w32, _scale(g32), b32.reshape(1, -1),
        w33, _scale(g33), b33.reshape(1, -1),
        wg1, _scale(gg1), bg1.reshape(1, -1),
        wg2, _scale(gg2), bg2.reshape(1, -1),
        Wc, bc.reshape(1, -1),
    )
    key_point_indices = jnp.zeros((B, 1024), jnp.int32)
    return logits, key_point_indices


# speedup vs baseline: 14.9816x; 1.0603x over previous
import functools

import jax
import jax.numpy as jnp
from jax import lax
from jax.experimental import pallas as pl
from jax.experimental.pallas import tpu as pltpu
from jax.experimental.pallas import tpu_sc as plsc

B = 16
N = 4096
EPS = 1e-5
_SC_CHUNK = 128


def _fps_body(npoint, n, px, py, pz, ox, oy, oz):
    b = px.shape[0]
    lane = lax.broadcasted_iota(jnp.int32, (b, n), 1)
    slot = lax.broadcasted_iota(jnp.int32, (b, npoint), 1)
    x = px[...]
    y = py[...]
    z = pz[...]

    def body(i, state):
        oxv, oyv, ozv, distance, far = state
        sel = lane == far
        cx = jnp.sum(jnp.where(sel, x, 0.0), axis=1, keepdims=True)
        cy = jnp.sum(jnp.where(sel, y, 0.0), axis=1, keepdims=True)
        cz = jnp.sum(jnp.where(sel, z, 0.0), axis=1, keepdims=True)
        hit = slot == i
        oxv = jnp.where(hit, cx, oxv)
        oyv = jnp.where(hit, cy, oyv)
        ozv = jnp.where(hit, cz, ozv)
        d = (x - cx) ** 2 + (y - cy) ** 2 + (z - cz) ** 2
        distance = jnp.minimum(distance, d)
        m = jnp.max(distance, axis=1, keepdims=True)
        far = jnp.min(jnp.where(distance == m, lane, n), axis=1, keepdims=True)
        return oxv, oyv, ozv, distance, far

    init = (
        jnp.zeros((b, npoint), jnp.float32),
        jnp.zeros((b, npoint), jnp.float32),
        jnp.zeros((b, npoint), jnp.float32),
        jnp.full((b, n), 1e10, jnp.float32),
        jnp.zeros((b, 1), jnp.int32),
    )
    oxv, oyv, ozv, _, _ = lax.fori_loop(0, npoint, body, init)
    ox[...] = oxv
    oy[...] = oyv
    oz[...] = ozv


def _fps(px, py, pz, npoint):
    b, n = px.shape
    out = jax.ShapeDtypeStruct((b, npoint), jnp.float32)
    return pl.pallas_call(
        functools.partial(_fps_body, npoint, n),
        out_shape=[out, out, out],
    )(px, py, pz)


def _ballq_body(n, s, ns, r2, px, py, pz, cx, cy, cz, out):
    x = px[0]
    y = py[0]
    z = pz[0]
    ax = cx[0]
    ay = cy[0]
    az = cz[0]
    lane = lax.broadcasted_iota(jnp.int32, (s, n), 1)
    slot = lax.broadcasted_iota(jnp.int32, (s, ns), 1)
    a2 = (ax * ax + ay * ay) + az * az
    b2 = (x * x + y * y) + z * z
    zc = jnp.zeros((s, 5), jnp.bfloat16)
    amat = jnp.concatenate(
        [ax.astype(jnp.bfloat16), ay.astype(jnp.bfloat16),
         az.astype(jnp.bfloat16), zc], axis=1)
    bmat = jnp.concatenate(
        [x.astype(jnp.bfloat16), y.astype(jnp.bfloat16),
         z.astype(jnp.bfloat16), jnp.zeros((5, n), jnp.bfloat16)], axis=0)
    e = jax.lax.dot_general(
        amat, bmat, (((1,), (0,)), ((), ())),
        preferred_element_type=jnp.float32)
    d2 = (a2 - 2.0 * e) + b2
    masked = jnp.where(d2 <= r2, lane, n)

    def body(k, state):
        gidx, prev = state
        cand = jnp.where(masked > prev, masked, n)
        mn = jnp.min(cand, axis=1, keepdims=True)
        gidx = jnp.where(slot == k, mn, gidx)
        return gidx, mn

    gidx0 = jnp.full((s, ns), n, jnp.int32)
    prev0 = jnp.full((s, 1), -1, jnp.int32)
    gidx, _ = lax.fori_loop(0, ns, body, (gidx0, prev0))
    first = gidx[:, :1]
    gidx = jnp.where(gidx == n, first, gidx)
    gidx = jnp.minimum(gidx, n - 1)
    base = pl.program_id(0) * n
    out[...] = (gidx + base)[None]


def _ballq(px, py, pz, cx, cy, cz, radius, ns):
    b, n = px.shape
    s = cx.shape[1]
    cxT = cx.reshape(b, s, 1)
    cyT = cy.reshape(b, s, 1)
    czT = cz.reshape(b, s, 1)
    pspec = pl.BlockSpec((1, 1, n), lambda i: (i, 0, 0))
    cspec = pl.BlockSpec((1, s, 1), lambda i: (i, 0, 0))
    return pl.pallas_call(
        functools.partial(_ballq_body, n, s, ns, radius * radius),
        grid=(b,),
        in_specs=[pspec, pspec, pspec, cspec, cspec, cspec],
        out_specs=pl.BlockSpec((1, s, ns), lambda i: (i, 0, 0)),
        out_shape=jax.ShapeDtypeStruct((b, s, ns), jnp.int32),
    )(px.reshape(b, 1, n), py.reshape(b, 1, n), pz.reshape(b, 1, n),
      cxT, cyT, czT)


def _gather_rows(table, idx):
    t = idx.shape[0]
    d = table.shape[1]
    info = plsc.get_sparse_core_info()
    nw = info.num_cores * info.num_subcores
    per_w = t // nw
    n_chunks = per_w // _SC_CHUNK
    nbuf = 4 if d <= 128 else 2
    idx3 = idx.reshape(nw, n_chunks, _SC_CHUNK)
    mesh = plsc.VectorSubcoreMesh(core_axis_name="c", subcore_axis_name="s")

    @functools.partial(
        pl.kernel,
        mesh=mesh,
        compiler_params=pltpu.CompilerParams(use_tc_tiling_on_sc=True),
        out_type=jax.ShapeDtypeStruct((t, d), jnp.float32),
        scratch_types=[
            pltpu.VMEM((n_chunks, _SC_CHUNK), jnp.int32),
            pltpu.VMEM((nbuf, _SC_CHUNK, d), jnp.float32),
            pltpu.SemaphoreType.DMA,
        ],
    )
    def k(table_hbm, idx_hbm, out_hbm, idx_v, rows_v, sem):
        wid = lax.axis_index("s") * info.num_cores + lax.axis_index("c")
        base = wid * per_w
        pltpu.sync_copy(idx_hbm.at[wid], idx_v)

        def body(g, carry):
            c0 = g * nbuf
            cps = [
                pltpu.async_copy(
                    table_hbm.at[idx_v.at[c0 + bb]], rows_v.at[bb], sem)
                for bb in range(nbuf)
            ]
            for bb in range(nbuf):
                cps[bb].wait()
            for bb in range(nbuf):
                pltpu.sync_copy(
                    rows_v.at[bb],
                    out_hbm.at[pl.ds(base + (c0 + bb) * _SC_CHUNK, _SC_CHUNK)],
                )
            return carry

        lax.fori_loop(0, n_chunks // nbuf, body, 0)

    return k(table, idx3)


def _bf(x):
    return x.astype(jnp.bfloat16)


def _dotb(a, w):
    return jnp.dot(_bf(a), _bf(w), preferred_element_type=jnp.float32)


def _sa_tail_body(sn, ns, g, cx, cy, cz, w1, s1, b1, w2, s2, b2, w3, s3, b3,
                  out):
    cp = g.shape[2]
    gx = g[0].reshape(sn, ns, cp)
    ax = cx[0]
    ay = cy[0]
    az = cz[0]
    cvec = jnp.concatenate([ax, ay, az, jnp.zeros((sn, cp - 3), jnp.float32)],
                           axis=1)
    xin = (gx - cvec[:, None, :]).reshape(sn * ns, cp)
    h1 = jnp.maximum(s1[...] * _dotb(xin, w1[...]) + b1[...], 0.0)
    h2 = jnp.maximum(s2[...] * _dotb(h1, w2[...]) + b2[...], 0.0)
    h3 = jnp.maximum(s3[...] * _dotb(h2, w3[...]) + b3[...], 0.0)
    out[...] = jnp.max(h3.reshape(sn, ns, h3.shape[1]), axis=1)[None]


def _sa_tail(g, cx, cy, cz, w1, s1, b1, w2, s2, b2, w3, s3, b3):
    b, tot, cp = g.shape
    sn = cx.shape[1]
    ns = tot // sn
    c3 = w3.shape[1]
    cspec = pl.BlockSpec((1, sn, 1), lambda i: (i, 0, 0))
    full = lambda *shape: pl.BlockSpec(shape, lambda i: tuple(0 for _ in shape))
    return pl.pallas_call(
        functools.partial(_sa_tail_body, sn, ns),
        grid=(b,),
        in_specs=[
            pl.BlockSpec((1, tot, cp), lambda i: (i, 0, 0)),
            cspec, cspec, cspec,
            full(*w1.shape), full(*s1.shape), full(*b1.shape),
            full(*w2.shape), full(*s2.shape), full(*b2.shape),
            full(*w3.shape), full(*s3.shape), full(*b3.shape),
        ],
        out_specs=pl.BlockSpec((1, sn, c3), lambda i: (i, 0, 0)),
        out_shape=jax.ShapeDtypeStruct((b, sn, c3), jnp.float32),
    )(g, cx.reshape(b, sn, 1), cy.reshape(b, sn, 1), cz.reshape(b, sn, 1),
      w1, s1, b1, w2, s2, b2, w3, s3, b3)


def _head_body(f2, cx, cy, cz, w31, s31, b31, w32, s32, b32, w33, s33,
               b33, wg1, sg1, bg1, wg2, sg2, bg2, wc, bcv, out):
    f = f2[0]
    x = jnp.concatenate([cx[0], cy[0], cz[0], f], axis=1)
    x1 = jnp.maximum(s31[...] * _dotb(x, w31[...]) + b31[...], 0.0)
    x2 = jnp.maximum(s32[...] * _dotb(x1, w32[...]) + b32[...], 0.0)
    x3 = jnp.maximum(s33[...] * _dotb(x2, w33[...]) + b33[...], 0.0)
    v = jnp.max(x3, axis=0, keepdims=True)
    g1 = jnp.maximum(sg1[...] * _dotb(v, wg1[...]) + bg1[...], 0.0)
    g2 = jnp.maximum(sg2[...] * _dotb(g1, wg2[...]) + bg2[...], 0.0)
    out[0] = _dotb(g2, wc[...]) + bcv[...]


def _head(f2, cx, cy, cz, w31, s31, b31, w32, s32, b32, w33, s33, b33,
          wg1, sg1, bg1, wg2, sg2, bg2, wc, bcv):
    b, s2n, c0 = f2.shape
    nc = wc.shape[1]
    cspec = pl.BlockSpec((1, s2n, 1), lambda i: (i, 0, 0))
    full = lambda *shape: pl.BlockSpec(shape, lambda i: tuple(0 for _ in shape))
    return pl.pallas_call(
        _head_body,
        grid=(b,),
        in_specs=[
            pl.BlockSpec((1, s2n, c0), lambda i: (i, 0, 0)),
            cspec, cspec, cspec,
            full(*w31.shape), full(*s31.shape),
            full(*b31.shape), full(*w32.shape), full(*s32.shape),
            full(*b32.shape), full(*w33.shape), full(*s33.shape),
            full(*b33.shape), full(*wg1.shape), full(*sg1.shape),
            full(*bg1.shape), full(*wg2.shape), full(*sg2.shape),
            full(*bg2.shape), full(*wc.shape), full(*bcv.shape),
        ],
        out_specs=pl.BlockSpec((1, 1, nc), lambda i: (i, 0, 0)),
        out_shape=jax.ShapeDtypeStruct((b, 1, nc), jnp.float32),
    )(f2, cx.reshape(b, s2n, 1), cy.reshape(b, s2n, 1), cz.reshape(b, s2n, 1),
      w31, s31, b31, w32, s32, b32, w33, s33, b33,
      wg1, sg1, bg1, wg2, sg2, bg2, wc, bcv).reshape(b, nc)


def _scale(gamma):
    return (gamma / jnp.sqrt(jnp.float32(1.0 + EPS))).reshape(1, -1)


def kernel(points, sa1, sa2, sa3, glob, Wc, bc):
    (w11, g11, b11), (w12, g12, b12), (w13, g13, b13) = sa1
    (w21, g21, b21), (w22, g22, b22), (w23, g23, b23) = sa2
    (w31, g31, b31), (w32, g32, b32), (w33, g33, b33) = sa3
    (wg1, gg1, bg1), (wg2, gg2, bg2) = glob

    px = points[:, 0, :]
    py = points[:, 1, :]
    pz = points[:, 2, :]

    pad = lambda a, r, c: jnp.pad(a, ((0, r - a.shape[0]), (0, c - a.shape[1])))
    ox1, oy1, oz1 = _fps(px, py, pz, 512)
    gidx1 = _ballq(px, py, pz, ox1, oy1, oz1, 0.2, 32)
    table1 = jnp.pad(jnp.stack([px, py, pz], axis=-1), ((0, 0), (0, 0), (0, 125)))
    g1 = _gather_rows(table1.reshape(B * N, 128), gidx1.reshape(-1))
    feat1 = _sa_tail(
        g1.reshape(B, 512 * 32, 128), ox1, oy1, oz1,
        pad(w11, 128, 64), _scale(g11), b11.reshape(1, -1),
        w12, _scale(g12), b12.reshape(1, -1),
        w13, _scale(g13), b13.reshape(1, -1),
    )

    ox2, oy2, oz2 = _fps(ox1, oy1, oz1, 128)
    gidx2 = _ballq(ox1, oy1, oz1, ox2, oy2, oz2, 0.4, 64)
    table2 = jnp.concatenate(
        [jnp.stack([ox1, oy1, oz1], axis=-1), feat1,
         jnp.zeros((B, 512, 125), jnp.float32)], axis=-1)
    g2 = _gather_rows(table2.reshape(B * 512, 256), gidx2.reshape(-1))
    feat2 = _sa_tail(
        g2.reshape(B, 128 * 64, 256), ox2, oy2, oz2,
        pad(w21, 256, 128), _scale(g21), b21.reshape(1, -1),
        w22, _scale(g22), b22.reshape(1, -1),
        w23, _scale(g23), b23.reshape(1, -1),
    )

    logits = _head(
        feat2, ox2, oy2, oz2,
        w31, _scale(g31), b31.reshape(1, -1),
        w32, _scale(g32), b32.reshape(1, -1),
        w33, _scale(g33), b33.reshape(1, -1),
        wg1, _scale(gg1), bg1.reshape(1, -1),
        wg2, _scale(gg2), bg2.reshape(1, -1),
        Wc, bc.reshape(1, -1),
    )
    key_point_indices = jnp.zeros((B, 1024), jnp.int32)
    return logits, key_point_indices
